# baseline (speedup 1.0000x reference)
import functools

import jax
import jax.numpy as jnp
from jax import lax
from jax.experimental import pallas as pl
from jax.experimental.pallas import tpu as pltpu
from jax.experimental.pallas import tpu_sc as plsc

BATCH = 16384
N_FIELDS = 26
EMBEDDING_DIM = 64

_B = BATCH * N_FIELDS
_NC = 2
_NS = 16
_NW = _NC * _NS
_B_PER_W = BATCH // _NW
_BBLK = 64
_NBLK = _B_PER_W // _BBLK
_BLK_ROWS = _BBLK * N_FIELDS
_CHUNK = 128
_N_CHUNKS = _BLK_ROWS // _CHUNK

_mesh = plsc.VectorSubcoreMesh(core_axis_name="c", subcore_axis_name="s")


@functools.partial(
    pl.kernel,
    mesh=_mesh,
    out_type=jax.ShapeDtypeStruct((N_FIELDS, EMBEDDING_DIM, BATCH), jnp.float32),
    scratch_types=[
        pltpu.VMEM((_BLK_ROWS,), jnp.int32),
        pltpu.VMEM((2, _CHUNK, EMBEDDING_DIM), jnp.float32),
        pltpu.VMEM((N_FIELDS, EMBEDDING_DIM, _BBLK), jnp.float32),
        pltpu.SemaphoreType.DMA((2,)),
        pltpu.SemaphoreType.DMA,
        pltpu.SemaphoreType.DMA,
    ],
    compiler_params=pltpu.CompilerParams(
        use_tc_tiling_on_sc=False, needs_layout_passes=False
    ),
)
def _sc_gather_t(idx_hbm, table_hbm, out_hbm, idx_v, rows_v, blk_v, gsems,
                 isem, bsem):
    wid = lax.axis_index("s") * _NC + lax.axis_index("c")
    d16 = lax.iota(jnp.int32, 16)

    def start_gather(c, b):
        pltpu.async_copy(
            table_hbm.at[idx_v.at[pl.ds(c * _CHUNK, _CHUNK)]],
            rows_v.at[b],
            gsems.at[b],
        )

    def wait_gather(b):
        pltpu.make_async_copy(
            table_hbm.at[idx_v.at[pl.ds(0, _CHUNK)]], rows_v.at[b],
            gsems.at[b],
        ).wait()

    def wait_block_store(b0):
        pltpu.make_async_copy(
            blk_v, out_hbm.at[:, :, pl.ds(b0, _BBLK)], bsem
        ).wait()

    def scatter_chunk(c, b):
        def row_body(i, carry):
            flat = c * _CHUNK + i
            b_loc = flat // N_FIELDS
            f = flat - b_loc * N_FIELDS
            f_vec = jnp.full((16,), f, jnp.int32)
            b_vec = jnp.full((16,), b_loc, jnp.int32)
            for j in range(EMBEDDING_DIM // 16):
                val = rows_v[b, i, pl.ds(j * 16, 16)]
                plsc.store_scatter(blk_v, [f_vec, d16 + (j * 16), b_vec], val)
            return carry

        lax.fori_loop(0, _CHUNK, row_body, 0)

    def block_body(blk, carry):
        base_b = wid * _B_PER_W + blk * _BBLK
        pltpu.async_copy(
            idx_hbm.at[pl.ds(base_b * N_FIELDS, _BLK_ROWS)], idx_v, isem
        ).wait()
        start_gather(0, 0)
        for c in range(_N_CHUNKS):
            if c + 1 < _N_CHUNKS:
                start_gather(c + 1, (c + 1) % 2)
            wait_gather(c % 2)

            @pl.when(jnp.logical_and(blk > 0, c == 0))
            def _():
                wait_block_store(base_b - _BBLK)

            scatter_chunk(c, c % 2)
        pltpu.async_copy(blk_v, out_hbm.at[:, :, pl.ds(base_b, _BBLK)], bsem)
        return carry

    lax.fori_loop(0, _NBLK, block_body, 0)
    wait_block_store(wid * _B_PER_W + (_NBLK - 1) * _BBLK)


def kernel(token_ids, weight):
    idx_flat = jnp.reshape(token_ids, (_B,)).astype(jnp.int32)
    out_t = _sc_gather_t(idx_flat, weight)
    return jnp.transpose(out_t, (2, 0, 1))
